# baseline (speedup 1.0000x reference)
import functools
import jax
import jax.numpy as jnp
from jax import lax
from jax.experimental import pallas as pl
from jax.experimental.pallas import tpu as pltpu, tpu_sc as plsc

_B, _S, _E = 4, 2048, 1024
_NW = 32
_SEQ_PER_W = _S // _NW
_POS_LEN = _SEQ_PER_W * _E
_CH_ROWS = 16
_CH = _CH_ROWS * _E
_CPB = _SEQ_PER_W // _CH_ROWS
_NCHUNK = _B * _CPB
_NBUF = 3
_VEC = 16


def _make_sc_kernel():
    mesh = plsc.VectorSubcoreMesh(core_axis_name="c", subcore_axis_name="s")

    @functools.partial(
        pl.kernel,
        mesh=mesh,
        out_type=jax.ShapeDtypeStruct((_B * _S * _E,), jnp.float32),
        scratch_types=[pltpu.VMEM((_CH,), jnp.float32)] * _NBUF
        + [pltpu.VMEM((_POS_LEN,), jnp.float32)]
        + [pltpu.SemaphoreType.DMA] * (2 * _NBUF + 1),
    )
    def k(x_hbm, pos_hbm, out_hbm, *rest):
        xbufs = rest[:_NBUF]
        pbuf = rest[_NBUF]
        sems = rest[_NBUF + 1:]
        lsems, ssems, psem = sems[:_NBUF], sems[_NBUF:2 * _NBUF], sems[-1]
        wid = lax.axis_index("s") * 2 + lax.axis_index("c")
        s0 = wid * _POS_LEN

        pos_load = pltpu.async_copy(pos_hbm.at[pl.ds(s0, _POS_LEN)], pbuf, psem)

        def off(ci):
            b, h = divmod(ci, _CPB)
            return b * (_S * _E) + s0 + h * _CH

        def load(ci):
            return pltpu.async_copy(
                x_hbm.at[pl.ds(off(ci), _CH)], xbufs[ci % _NBUF],
                lsems[ci % _NBUF],
            )

        loads, stores = {}, {}
        loads[0] = load(0)
        pos_load.wait()

        for ci in range(_NCHUNK):
            buf = ci % _NBUF
            if ci >= 2:
                stores[ci - 2].wait()
            if ci + 1 < _NCHUNK:
                loads[ci + 1] = load(ci + 1)
            loads[ci].wait()
            pbase = (ci % _CPB) * _CH

            xb = xbufs[buf]

            @plsc.parallel_loop(0, _CH, step=_VEC, unroll=8)
            def _(i):
                sl = pl.ds(i, _VEC)
                xb[sl] = xb[sl] + pbuf[pl.ds(pbase + i, _VEC)]

            stores[ci] = pltpu.async_copy(
                xb, out_hbm.at[pl.ds(off(ci), _CH)], ssems[buf]
            )
        stores[_NCHUNK - 1].wait()
        stores[_NCHUNK - 2].wait()

    return k


_sc_kernel = _make_sc_kernel()


def kernel(x, pos_table):
    B, S, E = x.shape
    out = _sc_kernel(x.reshape(-1), pos_table.reshape(-1))
    return out.reshape(B, S, E)


# speedup vs baseline: 2.4773x; 2.4773x over previous
import functools
import jax
import jax.numpy as jnp
from jax import lax
from jax.experimental import pallas as pl
from jax.experimental.pallas import tpu as pltpu, tpu_sc as plsc

_B, _S, _E = 4, 2048, 1024
_NW = 32
_SEQ_PER_W = _S // _NW
_CH_ROWS = 16
_CPB = _SEQ_PER_W // _CH_ROWS
_NCHUNK = _B * _CPB
_NBUF = 3
_VEC = 16
_CH = _CH_ROWS * _E


def _make_sc_kernel():
    mesh = plsc.VectorSubcoreMesh(core_axis_name="c", subcore_axis_name="s")

    @functools.partial(
        pl.kernel,
        mesh=mesh,
        out_type=jax.ShapeDtypeStruct((_B, _S, _E), jnp.float32),
        scratch_types=[pltpu.VMEM((_CH_ROWS, _E), jnp.float32)] * _NBUF
        + [pltpu.VMEM((_SEQ_PER_W, _E), jnp.float32)]
        + [pltpu.SemaphoreType.DMA] * (2 * _NBUF + 1),
    )
    def k(x_hbm, pos_hbm, out_hbm, *rest):
        xbufs = rest[:_NBUF]
        pbuf = rest[_NBUF]
        sems = rest[_NBUF + 1:]
        lsems, ssems, psem = sems[:_NBUF], sems[_NBUF:2 * _NBUF], sems[-1]
        wid = lax.axis_index("s") * 2 + lax.axis_index("c")
        r0 = wid * _SEQ_PER_W

        pos_load = pltpu.async_copy(
            pos_hbm.at[pl.ds(r0, _SEQ_PER_W), :], pbuf, psem
        )

        def rows(ci):
            b, h = divmod(ci, _CPB)
            return b, r0 + h * _CH_ROWS

        def load(ci):
            b, r = rows(ci)
            return pltpu.async_copy(
                x_hbm.at[b, pl.ds(r, _CH_ROWS), :], xbufs[ci % _NBUF],
                lsems[ci % _NBUF],
            )

        loads, stores = {}, {}
        loads[0] = load(0)
        pos_load.wait()

        for ci in range(_NCHUNK):
            buf = ci % _NBUF
            if ci >= 2:
                stores[ci - 2].wait()
            if ci + 1 < _NCHUNK:
                loads[ci + 1] = load(ci + 1)
            loads[ci].wait()
            h = ci % _CPB
            xb = xbufs[buf]

            @plsc.parallel_loop(0, _CH, step=_VEC, unroll=8)
            def _(i):
                r = lax.shift_right_logical(i, 10)
                c = pl.multiple_of(lax.bitwise_and(i, _E - 1), _VEC)
                sl = pl.ds(c, _VEC)
                xb[r, sl] = xb[r, sl] + pbuf[h * _CH_ROWS + r, sl]

            b, rr = rows(ci)
            stores[ci] = pltpu.async_copy(
                xb, out_hbm.at[b, pl.ds(rr, _CH_ROWS), :], ssems[buf]
            )
        stores[_NCHUNK - 1].wait()
        stores[_NCHUNK - 2].wait()

    return k


_sc_kernel = _make_sc_kernel()


def kernel(x, pos_table):
    return _sc_kernel(x, pos_table)


# speedup vs baseline: 2.4833x; 1.0024x over previous
import functools
import jax
import jax.numpy as jnp
from jax import lax
from jax.experimental import pallas as pl
from jax.experimental.pallas import tpu as pltpu, tpu_sc as plsc

_B, _S, _E = 4, 2048, 1024
_NW = 32
_SEQ_PER_W = _S // _NW
_CH_ROWS = 4
_NCHUNK = _SEQ_PER_W // _CH_ROWS
_NBUF = 3
_VEC = 16
_CHE = _CH_ROWS * _E


def _make_sc_kernel():
    mesh = plsc.VectorSubcoreMesh(core_axis_name="c", subcore_axis_name="s")

    @functools.partial(
        pl.kernel,
        mesh=mesh,
        out_type=jax.ShapeDtypeStruct((_B, _S, _E), jnp.float32),
        scratch_types=[pltpu.VMEM((_CH_ROWS, _E), jnp.float32)] * (_NBUF * _B)
        + [pltpu.VMEM((_SEQ_PER_W, _E), jnp.float32)]
        + [pltpu.SemaphoreType.DMA] * (2 * _NBUF + 1),
    )
    def k(x_hbm, pos_hbm, out_hbm, *rest):
        xbufs = rest[:_NBUF * _B]
        pbuf = rest[_NBUF * _B]
        sems = rest[_NBUF * _B + 1:]
        lsems, ssems, psem = sems[:_NBUF], sems[_NBUF:2 * _NBUF], sems[-1]
        wid = lax.axis_index("s") * 2 + lax.axis_index("c")
        r0 = wid * _SEQ_PER_W

        pos_load = pltpu.async_copy(
            pos_hbm.at[pl.ds(r0, _SEQ_PER_W), :], pbuf, psem
        )

        def load(ci):
            slot = ci % _NBUF
            r = r0 + ci * _CH_ROWS
            return [
                pltpu.async_copy(
                    x_hbm.at[b, pl.ds(r, _CH_ROWS), :],
                    xbufs[slot * _B + b], lsems[slot],
                )
                for b in range(_B)
            ]

        loads, stores = {}, {}
        loads[0] = load(0)
        pos_load.wait()

        for ci in range(_NCHUNK):
            slot = ci % _NBUF
            if ci >= 2:
                for d in stores[ci - 2]:
                    d.wait()
            if ci + 1 < _NCHUNK:
                loads[ci + 1] = load(ci + 1)
            for d in loads[ci]:
                d.wait()
            bufs = [xbufs[slot * _B + b] for b in range(_B)]
            prow0 = ci * _CH_ROWS

            @plsc.parallel_loop(0, _CHE, step=_VEC, unroll=4)
            def _(i):
                rr = lax.shift_right_logical(i, 10)
                c = pl.multiple_of(lax.bitwise_and(i, _E - 1), _VEC)
                sl = pl.ds(c, _VEC)
                pv = pbuf[prow0 + rr, sl]
                for b in range(_B):
                    bufs[b][rr, sl] = bufs[b][rr, sl] + pv

            r = r0 + ci * _CH_ROWS
            stores[ci] = [
                pltpu.async_copy(
                    bufs[b], out_hbm.at[b, pl.ds(r, _CH_ROWS), :], ssems[slot]
                )
                for b in range(_B)
            ]
        for ci in (_NCHUNK - 2, _NCHUNK - 1):
            for d in stores[ci]:
                d.wait()

    return k


_sc_kernel = _make_sc_kernel()


def kernel(x, pos_table):
    return _sc_kernel(x, pos_table)
